# baseline (speedup 1.0000x reference)
import functools

import jax
import jax.numpy as jnp
from jax import lax
from jax.experimental import pallas as pl
from jax.experimental.pallas import tpu as pltpu
from jax.experimental.pallas import tpu_sc as plsc

N = 10000
E = 320000
D_H = 128
DHALF = D_H // 2
D_OUT = 40

NC = 2
NS = 16

CB = 128
EPT = E // NS
KCH = 162
EPT_PAD = KCH * CB
E_PAD = EPT_PAD * NS
NBUF = 6
PF = 3

NP = 10016
RPT = NP // NS


def _sc_agg_body(src0_hbm, src1_hbm, dst_hbm, embcat_hbm, zeros_hbm,
                 out0, out1,
                 src_v, dst_v, rows_v, agg_sh,
                 gs0, gs1, gs2, gs3, gs4, gs5, ss0, ss1, ss2, ss3, ss4, ss5):
    cid = lax.axis_index("c")
    sid = lax.axis_index("s")
    gsems = (gs0, gs1, gs2, gs3, gs4, gs5)
    ssems = (ss0, ss1, ss2, ss3, ss4, ss5)

    pltpu.sync_copy(zeros_hbm.at[pl.ds(sid * RPT, RPT)],
                    agg_sh.at[pl.ds(sid * RPT, RPT)])

    @pl.when(cid == 0)
    def _():
        pltpu.sync_copy(src0_hbm.at[sid], src_v)

    @pl.when(cid == 1)
    def _():
        pltpu.sync_copy(src1_hbm.at[sid], src_v)

    pltpu.sync_copy(dst_hbm.at[sid], dst_v)
    plsc.subcore_barrier()

    def start_gather(j, b):
        pltpu.async_copy(embcat_hbm.at[src_v.at[j]], rows_v.at[b], gsems[b])

    def wait_dma(sems, b):
        pltpu.make_async_copy(embcat_hbm.at[pl.ds(0, CB)], rows_v.at[b],
                              sems[b]).wait()

    for b in range(PF):
        start_gather(b, b)

    def round_body(g, carry):
        for b in range(NBUF):
            j = g * NBUF + b
            bn = (b + PF) % NBUF

            def retire_and_prefetch():
                wait_dma(ssems, bn)
                start_gather(j + PF, bn)

            if b < PF:
                @pl.when(g > 0)
                def _():
                    retire_and_prefetch()

                @pl.when(g == 0)
                def _():
                    start_gather(j + PF, bn)
            else:
                @pl.when(g < KCH // NBUF - 1)
                def _():
                    retire_and_prefetch()

            wait_dma(gsems, b)
            pltpu.async_copy(rows_v.at[b], agg_sh.at[dst_v.at[j]], ssems[b],
                             add=True)
        return carry

    lax.fori_loop(0, KCH // NBUF, round_body, 0)
    for b in range(NBUF):
        wait_dma(ssems, b)
    plsc.subcore_barrier()

    @pl.when(cid == 0)
    def _():
        pltpu.sync_copy(agg_sh.at[pl.ds(sid * RPT, RPT)],
                        out0.at[pl.ds(sid * RPT, RPT)])

    @pl.when(cid == 1)
    def _():
        pltpu.sync_copy(agg_sh.at[pl.ds(sid * RPT, RPT)],
                        out1.at[pl.ds(sid * RPT, RPT)])


_sc_agg = functools.partial(
    pl.kernel,
    mesh=plsc.VectorSubcoreMesh(core_axis_name="c", subcore_axis_name="s"),
    compiler_params=pltpu.CompilerParams(use_tc_tiling_on_sc=False),
    out_type=[jax.ShapeDtypeStruct((NP, DHALF), jnp.float32),
              jax.ShapeDtypeStruct((NP, DHALF), jnp.float32)],
    scratch_types=(
        [pltpu.VMEM((KCH, CB), jnp.int32),
         pltpu.VMEM((KCH, CB), jnp.int32),
         pltpu.VMEM((NBUF, CB, DHALF), jnp.float32),
         pltpu.VMEM_SHARED((NP, DHALF), jnp.float32)]
        + [pltpu.SemaphoreType.DMA] * (2 * NBUF)
    ),
)(_sc_agg_body)


BN = 1000


def _tc_body(p0_ref, p1_ref, x_ref, wl_ref, wh_ref, om_ref, b_ref, hw_ref,
             hb_ref, out_ref):
    dn = (((1,), (1,)), ((), ()))
    t = lax.dot_general(p0_ref[...], wl_ref[...], dn,
                        preferred_element_type=jnp.float32)
    t += lax.dot_general(p1_ref[...], wh_ref[...], dn,
                         preferred_element_type=jnp.float32)
    t += lax.dot_general(x_ref[...], om_ref[...], dn,
                         preferred_element_type=jnp.float32)
    h = jnp.maximum(t + b_ref[...], 0.0)
    o = lax.dot_general(h, hw_ref[...], dn, preferred_element_type=jnp.float32)
    o += hb_ref[...]
    m = jnp.max(o, axis=-1, keepdims=True)
    ex = jnp.exp(o - m)
    s = jnp.sum(ex, axis=-1, keepdims=True)
    out_ref[...] = o - m - jnp.log(s)


def _tc_head(p0, p1, x, Wl, Wh, Omega, b, head_W, head_b):
    grid = (N // BN,)
    half_spec = pl.BlockSpec((BN, DHALF), lambda i: (i, 0))
    return pl.pallas_call(
        _tc_body,
        grid=grid,
        in_specs=[
            half_spec,
            half_spec,
            pl.BlockSpec((BN, D_H), lambda i: (i, 0)),
            pl.BlockSpec((D_H, DHALF), lambda i: (0, 0)),
            pl.BlockSpec((D_H, DHALF), lambda i: (0, 0)),
            pl.BlockSpec((D_H, D_H), lambda i: (0, 0)),
            pl.BlockSpec((1, D_H), lambda i: (0, 0)),
            pl.BlockSpec((D_OUT, D_H), lambda i: (0, 0)),
            pl.BlockSpec((1, D_OUT), lambda i: (0, 0)),
        ],
        out_specs=pl.BlockSpec((BN, D_OUT), lambda i: (i, 0)),
        out_shape=jax.ShapeDtypeStruct((N, D_OUT), jnp.float32),
    )(p0, p1, x, Wl, Wh, Omega, b, head_W, head_b)


@jax.jit
def kernel(node_index, x, edge_index, emb_table, W, Omega, b, head_W, head_b):
    del node_index
    src = edge_index[0]
    dst = edge_index[1]
    pad = E_PAD - E
    src_p = jnp.concatenate([src, jnp.zeros((pad,), jnp.int32)])
    dst_p = jnp.concatenate([dst, jnp.full((pad,), N, jnp.int32)])
    src0 = src_p.reshape(NS, KCH, CB)
    src1 = src0 + N
    dst_p = dst_p.reshape(NS, KCH, CB)
    emb_cat = jnp.concatenate([emb_table[:, :DHALF], emb_table[:, DHALF:]],
                              axis=0)
    zeros = jnp.zeros((NP, DHALF), jnp.float32)

    p0, p1 = _sc_agg(src0, src1, dst_p, emb_cat, zeros)
    return _tc_head(p0[:N], p1[:N], x, W[:, :DHALF], W[:, DHALF:], Omega,
                    b.reshape(1, D_H), head_W, head_b.reshape(1, D_OUT))


# speedup vs baseline: 1.2614x; 1.2614x over previous
import functools

import jax
import jax.numpy as jnp
from jax import lax
from jax.experimental import pallas as pl
from jax.experimental.pallas import tpu as pltpu
from jax.experimental.pallas import tpu_sc as plsc

N = 10000
E = 320000
D_H = 128
D_OUT = 40

NC = 2
NS = 16
NW = NC * NS

CB = 96
EPT = E // NW
KCH = 106
EPT_PAD = KCH * CB
E_PAD = EPT_PAD * NW
NBUF = 2
PF = 1

NP = 10016
RPT = NP // NS


def _sc_agg_body(src_hbm, dst_hbm, emb_hbm, zeros_hbm, out0, out1,
                 src_v, dst_v, rows_v, agg_sh, gs0, gs1, ss0, ss1):
    cid = lax.axis_index("c")
    sid = lax.axis_index("s")
    wid = cid * NS + sid
    gsems = (gs0, gs1)
    ssems = (ss0, ss1)

    pltpu.sync_copy(zeros_hbm.at[pl.ds(sid * RPT, RPT)],
                    agg_sh.at[pl.ds(sid * RPT, RPT)])
    pltpu.sync_copy(src_hbm.at[wid], src_v)
    pltpu.sync_copy(dst_hbm.at[wid], dst_v)
    plsc.subcore_barrier()

    def start_gather(j, b):
        pltpu.async_copy(emb_hbm.at[src_v.at[j]], rows_v.at[b], gsems[b])

    def wait_dma(sems, b):
        pltpu.make_async_copy(emb_hbm.at[pl.ds(0, CB)], rows_v.at[b],
                              sems[b]).wait()

    start_gather(0, 0)

    def round_body(g, carry):
        for b in range(NBUF):
            j = g * NBUF + b

            bn = (b + PF) % NBUF

            def retire_and_prefetch():
                wait_dma(ssems, bn)
                start_gather(j + PF, bn)

            if b < PF:
                @pl.when(g > 0)
                def _():
                    retire_and_prefetch()

                @pl.when(g == 0)
                def _():
                    start_gather(j + PF, bn)
            else:
                @pl.when(g < KCH // NBUF - 1)
                def _():
                    retire_and_prefetch()

            wait_dma(gsems, b)
            pltpu.async_copy(rows_v.at[b], agg_sh.at[dst_v.at[j]],
                             ssems[b], add=True)
        return carry

    lax.fori_loop(0, KCH // NBUF, round_body, 0)
    for b in range(NBUF):
        wait_dma(ssems, b)
    plsc.subcore_barrier()

    @pl.when(cid == 0)
    def _():
        pltpu.sync_copy(agg_sh.at[pl.ds(sid * RPT, RPT)],
                        out0.at[pl.ds(sid * RPT, RPT)])

    @pl.when(cid == 1)
    def _():
        pltpu.sync_copy(agg_sh.at[pl.ds(sid * RPT, RPT)],
                        out1.at[pl.ds(sid * RPT, RPT)])


_sc_agg = functools.partial(
    pl.kernel,
    mesh=plsc.VectorSubcoreMesh(core_axis_name="c", subcore_axis_name="s"),
    compiler_params=pltpu.CompilerParams(use_tc_tiling_on_sc=False),
    out_type=[jax.ShapeDtypeStruct((NP, D_H), jnp.float32),
              jax.ShapeDtypeStruct((NP, D_H), jnp.float32)],
    scratch_types=(
        [pltpu.VMEM((KCH, CB), jnp.int32),
         pltpu.VMEM((KCH, CB), jnp.int32),
         pltpu.VMEM((NBUF, CB, D_H), jnp.float32),
         pltpu.VMEM_SHARED((NP, D_H), jnp.float32)]
        + [pltpu.SemaphoreType.DMA] * (2 * NBUF)
    ),
)(_sc_agg_body)


BN = 1000


def _tc_body(p0_ref, p1_ref, x_ref, w_ref, om_ref, b_ref, hw_ref, hb_ref,
             out_ref):
    agg = p0_ref[...] + p1_ref[...]
    dn = (((1,), (1,)), ((), ()))
    t = lax.dot_general(agg, w_ref[...], dn, preferred_element_type=jnp.float32)
    t += lax.dot_general(x_ref[...], om_ref[...], dn,
                         preferred_element_type=jnp.float32)
    h = jnp.maximum(t + b_ref[...], 0.0)
    o = lax.dot_general(h, hw_ref[...], dn, preferred_element_type=jnp.float32)
    o += hb_ref[...]
    m = jnp.max(o, axis=-1, keepdims=True)
    ex = jnp.exp(o - m)
    s = jnp.sum(ex, axis=-1, keepdims=True)
    out_ref[...] = o - m - jnp.log(s)


def _tc_head(p0, p1, x, W, Omega, b, head_W, head_b):
    grid = (N // BN,)
    row_spec = pl.BlockSpec((BN, D_H), lambda i: (i, 0))
    return pl.pallas_call(
        _tc_body,
        grid=grid,
        in_specs=[
            row_spec,
            row_spec,
            row_spec,
            pl.BlockSpec((D_H, D_H), lambda i: (0, 0)),
            pl.BlockSpec((D_H, D_H), lambda i: (0, 0)),
            pl.BlockSpec((1, D_H), lambda i: (0, 0)),
            pl.BlockSpec((D_OUT, D_H), lambda i: (0, 0)),
            pl.BlockSpec((1, D_OUT), lambda i: (0, 0)),
        ],
        out_specs=pl.BlockSpec((BN, D_OUT), lambda i: (i, 0)),
        out_shape=jax.ShapeDtypeStruct((N, D_OUT), jnp.float32),
    )(p0, p1, x, W, Omega, b, head_W, head_b)


@jax.jit
def kernel(node_index, x, edge_index, emb_table, W, Omega, b, head_W, head_b):
    del node_index
    src = edge_index[0]
    dst = edge_index[1]
    pad = E_PAD - E
    src_p = jnp.concatenate([src, jnp.zeros((pad,), jnp.int32)])
    dst_p = jnp.concatenate([dst, jnp.full((pad,), N, jnp.int32)])
    src_p = src_p.reshape(NW, KCH, CB)
    dst_p = dst_p.reshape(NW, KCH, CB)
    zeros = jnp.zeros((NP, D_H), jnp.float32)

    p0, p1 = _sc_agg(src_p, dst_p, emb_table, zeros)
    return _tc_head(p0[:N], p1[:N], x, W, Omega, b.reshape(1, D_H),
                    head_W, head_b.reshape(1, D_OUT))


# speedup vs baseline: 1.3836x; 1.0969x over previous
import functools

import jax
import jax.numpy as jnp
from jax import lax
from jax.experimental import pallas as pl
from jax.experimental.pallas import tpu as pltpu
from jax.experimental.pallas import tpu_sc as plsc

N = 10000
E = 320000
D_H = 128
D_OUT = 40

NC = 2
NS = 16
NW = NC * NS

CB = 128
EPT = E // NW
KCH = 79
EPT_PAD = KCH * CB
E_PAD = EPT_PAD * NW

NP = 10016
RPT = NP // NS


def _sc_agg_body(src_hbm, dst_hbm, emb_hbm, zeros_hbm, out0, out1,
                 src_v, dst_v, rows_v, agg_sh, sem):
    cid = lax.axis_index("c")
    sid = lax.axis_index("s")
    wid = cid * NS + sid
    pltpu.sync_copy(zeros_hbm.at[pl.ds(sid * RPT, RPT)],
                    agg_sh.at[pl.ds(sid * RPT, RPT)])
    pltpu.sync_copy(src_hbm.at[wid], src_v)
    pltpu.sync_copy(dst_hbm.at[wid], dst_v)
    plsc.subcore_barrier()

    def chunk(j, carry):
        pltpu.async_copy(emb_hbm.at[src_v.at[j]], rows_v, sem).wait()
        pltpu.sync_copy(rows_v, agg_sh.at[dst_v.at[j]], add=True)
        return carry

    lax.fori_loop(0, KCH, chunk, 0)
    plsc.subcore_barrier()

    @pl.when(cid == 0)
    def _():
        pltpu.sync_copy(agg_sh.at[pl.ds(sid * RPT, RPT)],
                        out0.at[pl.ds(sid * RPT, RPT)])

    @pl.when(cid == 1)
    def _():
        pltpu.sync_copy(agg_sh.at[pl.ds(sid * RPT, RPT)],
                        out1.at[pl.ds(sid * RPT, RPT)])


_sc_agg = functools.partial(
    pl.kernel,
    mesh=plsc.VectorSubcoreMesh(core_axis_name="c", subcore_axis_name="s"),
    compiler_params=pltpu.CompilerParams(use_tc_tiling_on_sc=False),
    out_type=[jax.ShapeDtypeStruct((NP, D_H), jnp.float32),
              jax.ShapeDtypeStruct((NP, D_H), jnp.float32)],
    scratch_types=(
        [pltpu.VMEM((KCH, CB), jnp.int32),
         pltpu.VMEM((KCH, CB), jnp.int32),
         pltpu.VMEM((CB, D_H), jnp.float32),
         pltpu.VMEM_SHARED((NP, D_H), jnp.float32),
         pltpu.SemaphoreType.DMA]
    ),
)(_sc_agg_body)


BN = 1000


def _tc_body(p0_ref, p1_ref, x_ref, w_ref, om_ref, b_ref, hw_ref, hb_ref,
             out_ref):
    agg = p0_ref[...] + p1_ref[...]
    dn = (((1,), (1,)), ((), ()))
    t = lax.dot_general(agg, w_ref[...], dn, preferred_element_type=jnp.float32)
    t += lax.dot_general(x_ref[...], om_ref[...], dn,
                         preferred_element_type=jnp.float32)
    h = jnp.maximum(t + b_ref[...], 0.0)
    o = lax.dot_general(h, hw_ref[...], dn, preferred_element_type=jnp.float32)
    o += hb_ref[...]
    m = jnp.max(o, axis=-1, keepdims=True)
    ex = jnp.exp(o - m)
    s = jnp.sum(ex, axis=-1, keepdims=True)
    out_ref[...] = o - m - jnp.log(s)


def _tc_head(p0, p1, x, W, Omega, b, head_W, head_b):
    grid = (N // BN,)
    row_spec = pl.BlockSpec((BN, D_H), lambda i: (i, 0))
    return pl.pallas_call(
        _tc_body,
        grid=grid,
        in_specs=[
            row_spec,
            row_spec,
            row_spec,
            pl.BlockSpec((D_H, D_H), lambda i: (0, 0)),
            pl.BlockSpec((D_H, D_H), lambda i: (0, 0)),
            pl.BlockSpec((1, D_H), lambda i: (0, 0)),
            pl.BlockSpec((D_OUT, D_H), lambda i: (0, 0)),
            pl.BlockSpec((1, D_OUT), lambda i: (0, 0)),
        ],
        out_specs=pl.BlockSpec((BN, D_OUT), lambda i: (i, 0)),
        out_shape=jax.ShapeDtypeStruct((N, D_OUT), jnp.float32),
    )(p0, p1, x, W, Omega, b, head_W, head_b)


@jax.jit
def kernel(node_index, x, edge_index, emb_table, W, Omega, b, head_W, head_b):
    del node_index
    src = edge_index[0]
    dst = edge_index[1]
    pad = E_PAD - E
    src_p = jnp.concatenate([src, jnp.zeros((pad,), jnp.int32)])
    dst_p = jnp.concatenate([dst, jnp.full((pad,), N, jnp.int32)])
    src_p = src_p.reshape(NW, KCH, CB)
    dst_p = dst_p.reshape(NW, KCH, CB)
    zeros = jnp.zeros((NP, D_H), jnp.float32)

    p0, p1 = _sc_agg(src_p, dst_p, emb_table, zeros)
    return _tc_head(p0[:N], p1[:N], x, W, Omega, b.reshape(1, D_H),
                    head_W, head_b.reshape(1, D_OUT))


# speedup vs baseline: 1.6654x; 1.2037x over previous
import functools

import jax
import jax.numpy as jnp
from jax import lax
from jax.experimental import pallas as pl
from jax.experimental.pallas import tpu as pltpu
from jax.experimental.pallas import tpu_sc as plsc

N = 10000
E = 320000
D_H = 128
D_OUT = 40

NC = 2
NS = 16
NW = NC * NS

CB = 240
EPT = E // NW
KCH = 42
EPT_PAD = KCH * CB
E_PAD = EPT_PAD * NW

NP = 10016
RPT = NP // NS


def _sc_agg_body(src_hbm, dst_hbm, emb_hbm, zeros_hbm, out0, out1,
                 src_v, dst_v, rows_v, agg_sh, sem):
    cid = lax.axis_index("c")
    sid = lax.axis_index("s")
    wid = cid * NS + sid
    pltpu.sync_copy(zeros_hbm.at[pl.ds(sid * RPT, RPT)],
                    agg_sh.at[pl.ds(sid * RPT, RPT)])
    pltpu.sync_copy(src_hbm.at[wid], src_v)
    pltpu.sync_copy(dst_hbm.at[wid], dst_v)
    plsc.subcore_barrier()

    def chunk(j, carry):
        pltpu.async_copy(emb_hbm.at[src_v.at[j]], rows_v, sem).wait()
        pltpu.sync_copy(rows_v, agg_sh.at[dst_v.at[j]], add=True)
        return carry

    lax.fori_loop(0, KCH, chunk, 0)
    plsc.subcore_barrier()

    @pl.when(cid == 0)
    def _():
        pltpu.sync_copy(agg_sh.at[pl.ds(sid * RPT, RPT)],
                        out0.at[pl.ds(sid * RPT, RPT)])

    @pl.when(cid == 1)
    def _():
        pltpu.sync_copy(agg_sh.at[pl.ds(sid * RPT, RPT)],
                        out1.at[pl.ds(sid * RPT, RPT)])


_sc_agg = functools.partial(
    pl.kernel,
    mesh=plsc.VectorSubcoreMesh(core_axis_name="c", subcore_axis_name="s"),
    compiler_params=pltpu.CompilerParams(use_tc_tiling_on_sc=False),
    out_type=[jax.ShapeDtypeStruct((NP, D_H), jnp.float32),
              jax.ShapeDtypeStruct((NP, D_H), jnp.float32)],
    scratch_types=(
        [pltpu.VMEM((KCH, CB), jnp.int32),
         pltpu.VMEM((KCH, CB), jnp.int32),
         pltpu.VMEM((CB, D_H), jnp.float32),
         pltpu.VMEM_SHARED((NP, D_H), jnp.float32),
         pltpu.SemaphoreType.DMA]
    ),
)(_sc_agg_body)


BN = 1000


def _tc_body(p0_ref, p1_ref, x_ref, w_ref, om_ref, b_ref, hw_ref, hb_ref,
             out_ref):
    agg = p0_ref[...] + p1_ref[...]
    dn = (((1,), (1,)), ((), ()))
    t = lax.dot_general(agg, w_ref[...], dn, preferred_element_type=jnp.float32)
    t += lax.dot_general(x_ref[...], om_ref[...], dn,
                         preferred_element_type=jnp.float32)
    h = jnp.maximum(t + b_ref[...], 0.0)
    o = lax.dot_general(h, hw_ref[...], dn, preferred_element_type=jnp.float32)
    o += hb_ref[...]
    m = jnp.max(o, axis=-1, keepdims=True)
    ex = jnp.exp(o - m)
    s = jnp.sum(ex, axis=-1, keepdims=True)
    out_ref[...] = o - m - jnp.log(s)


def _tc_head(p0, p1, x, W, Omega, b, head_W, head_b):
    grid = (N // BN,)
    row_spec = pl.BlockSpec((BN, D_H), lambda i: (i, 0))
    return pl.pallas_call(
        _tc_body,
        grid=grid,
        in_specs=[
            row_spec,
            row_spec,
            row_spec,
            pl.BlockSpec((D_H, D_H), lambda i: (0, 0)),
            pl.BlockSpec((D_H, D_H), lambda i: (0, 0)),
            pl.BlockSpec((1, D_H), lambda i: (0, 0)),
            pl.BlockSpec((D_OUT, D_H), lambda i: (0, 0)),
            pl.BlockSpec((1, D_OUT), lambda i: (0, 0)),
        ],
        out_specs=pl.BlockSpec((BN, D_OUT), lambda i: (i, 0)),
        out_shape=jax.ShapeDtypeStruct((N, D_OUT), jnp.float32),
    )(p0, p1, x, W, Omega, b, head_W, head_b)


@jax.jit
def kernel(node_index, x, edge_index, emb_table, W, Omega, b, head_W, head_b):
    del node_index
    src = edge_index[0]
    dst = edge_index[1]
    pad = E_PAD - E
    src_p = jnp.concatenate([src, jnp.zeros((pad,), jnp.int32)])
    dst_p = jnp.concatenate([dst, jnp.full((pad,), N, jnp.int32)])
    src_p = src_p.reshape(NW, KCH, CB)
    dst_p = dst_p.reshape(NW, KCH, CB)
    zeros = jnp.zeros((NP, D_H), jnp.float32)

    p0, p1 = _sc_agg(src_p, dst_p, emb_table, zeros)
    return _tc_head(p0[:N], p1[:N], x, W, Omega, b.reshape(1, D_H),
                    head_W, head_b.reshape(1, D_OUT))


# speedup vs baseline: 2.2679x; 1.3618x over previous
import functools

import jax
import jax.numpy as jnp
from jax import lax
from jax.experimental import pallas as pl
from jax.experimental.pallas import tpu as pltpu
from jax.experimental.pallas import tpu_sc as plsc

N = 10000
E = 320000
D_H = 128
D_OUT = 40

NC = 2
NS = 16
NW = NC * NS

CB = 200
K0 = 63
K1 = 37
NR = (K0 + K1) * NS

NP = N
RPT = NP // NS


def _sc_agg_body(src_hbm, dst_hbm, emb_hbm, zeros_hbm, out0, out1,
                 src_v, dst_v, rows_v, agg_sh, sem):
    cid = lax.axis_index("c")
    sid = lax.axis_index("s")
    pltpu.sync_copy(zeros_hbm.at[pl.ds(sid * RPT, RPT)],
                    agg_sh.at[pl.ds(sid * RPT, RPT)])
    @pl.when(cid == 0)
    def _():
        pltpu.sync_copy(src_hbm.at[pl.ds(sid * K0, K0)], src_v)
        pltpu.sync_copy(dst_hbm.at[pl.ds(sid * K0, K0)], dst_v)

    @pl.when(cid == 1)
    def _():
        pltpu.sync_copy(src_hbm.at[pl.ds(NS * K0 + sid * K1, K1)],
                        src_v.at[pl.ds(0, K1)])
        pltpu.sync_copy(dst_hbm.at[pl.ds(NS * K0 + sid * K1, K1)],
                        dst_v.at[pl.ds(0, K1)])

    plsc.subcore_barrier()

    def chunk(j, carry):
        pltpu.async_copy(emb_hbm.at[src_v.at[j]], rows_v, sem).wait()
        pltpu.sync_copy(rows_v, agg_sh.at[dst_v.at[j]], add=True)
        return carry

    nk = jnp.where(cid == 0, K0, K1)
    lax.fori_loop(0, nk, chunk, 0)
    plsc.subcore_barrier()

    @pl.when(cid == 0)
    def _():
        pltpu.sync_copy(agg_sh.at[pl.ds(sid * RPT, RPT)],
                        out0.at[pl.ds(sid * RPT, RPT)])

    @pl.when(cid == 1)
    def _():
        pltpu.sync_copy(agg_sh.at[pl.ds(sid * RPT, RPT)],
                        out1.at[pl.ds(sid * RPT, RPT)])


_sc_agg = functools.partial(
    pl.kernel,
    mesh=plsc.VectorSubcoreMesh(core_axis_name="c", subcore_axis_name="s"),
    compiler_params=pltpu.CompilerParams(use_tc_tiling_on_sc=False),
    out_type=[jax.ShapeDtypeStruct((NP, D_H), jnp.float32),
              jax.ShapeDtypeStruct((NP, D_H), jnp.float32)],
    scratch_types=(
        [pltpu.VMEM((K0, CB), jnp.int32),
         pltpu.VMEM((K0, CB), jnp.int32),
         pltpu.VMEM((CB, D_H), jnp.float32),
         pltpu.VMEM_SHARED((NP, D_H), jnp.float32),
         pltpu.SemaphoreType.DMA]
    ),
)(_sc_agg_body)


BN = 1000


def _tc_body(p0_ref, p1_ref, x_ref, w_ref, om_ref, b_ref, hw_ref, hb_ref,
             out_ref):
    agg = p0_ref[...] + p1_ref[...]
    dn = (((1,), (1,)), ((), ()))
    t = lax.dot_general(agg, w_ref[...], dn, preferred_element_type=jnp.float32)
    t += lax.dot_general(x_ref[...], om_ref[...], dn,
                         preferred_element_type=jnp.float32)
    h = jnp.maximum(t + b_ref[...], 0.0)
    o = lax.dot_general(h, hw_ref[...], dn, preferred_element_type=jnp.float32)
    o += hb_ref[...]
    m = jnp.max(o, axis=-1, keepdims=True)
    ex = jnp.exp(o - m)
    s = jnp.sum(ex, axis=-1, keepdims=True)
    out_ref[...] = o - m - jnp.log(s)


def _tc_head(p0, p1, x, W, Omega, b, head_W, head_b):
    grid = (N // BN,)
    row_spec = pl.BlockSpec((BN, D_H), lambda i: (i, 0))
    return pl.pallas_call(
        _tc_body,
        grid=grid,
        in_specs=[
            row_spec,
            row_spec,
            row_spec,
            pl.BlockSpec((D_H, D_H), lambda i: (0, 0)),
            pl.BlockSpec((D_H, D_H), lambda i: (0, 0)),
            pl.BlockSpec((1, D_H), lambda i: (0, 0)),
            pl.BlockSpec((D_OUT, D_H), lambda i: (0, 0)),
            pl.BlockSpec((1, D_OUT), lambda i: (0, 0)),
        ],
        out_specs=pl.BlockSpec((BN, D_OUT), lambda i: (i, 0)),
        out_shape=jax.ShapeDtypeStruct((N, D_OUT), jnp.float32),
    )(p0, p1, x, W, Omega, b, head_W, head_b)


@jax.jit
def kernel(node_index, x, edge_index, emb_table, W, Omega, b, head_W, head_b):
    del node_index
    src = edge_index[0]
    dst = edge_index[1]
    src_p = src.reshape(NR, CB)
    dst_p = dst.reshape(NR, CB)
    zeros = jnp.zeros((NP, D_H), jnp.float32)

    p0, p1 = _sc_agg(src_p, dst_p, emb_table, zeros)
    return _tc_head(p0, p1, x, W, Omega, b.reshape(1, D_H),
                    head_W, head_b.reshape(1, D_OUT))


# speedup vs baseline: 2.4013x; 1.0588x over previous
import functools

import jax
import jax.numpy as jnp
from jax import lax
from jax.experimental import pallas as pl
from jax.experimental.pallas import tpu as pltpu
from jax.experimental.pallas import tpu_sc as plsc

N = 10000
E = 320000
D_H = 128
D_OUT = 40

NC = 2
NS = 16
NW = NC * NS

CB = 200
K0 = 63
K1 = 37
NR = (K0 + K1) * NS

NP = N
RPT = NP // NS


def _sc_agg_body(edge_hbm, emb_hbm, zeros_hbm, out0, out1,
                 src_v, dst_v, rows_v, agg_sh, sem):
    cid = lax.axis_index("c")
    sid = lax.axis_index("s")
    pltpu.sync_copy(zeros_hbm.at[pl.ds(sid * RPT, RPT)],
                    agg_sh.at[pl.ds(sid * RPT, RPT)])
    @pl.when(cid == 0)
    def _():
        pltpu.sync_copy(edge_hbm.at[0, pl.ds(sid * K0, K0)], src_v)
        pltpu.sync_copy(edge_hbm.at[1, pl.ds(sid * K0, K0)], dst_v)

    @pl.when(cid == 1)
    def _():
        pltpu.sync_copy(edge_hbm.at[0, pl.ds(NS * K0 + sid * K1, K1)],
                        src_v.at[pl.ds(0, K1)])
        pltpu.sync_copy(edge_hbm.at[1, pl.ds(NS * K0 + sid * K1, K1)],
                        dst_v.at[pl.ds(0, K1)])

    plsc.subcore_barrier()

    def chunk(j, carry):
        pltpu.async_copy(emb_hbm.at[src_v.at[j]], rows_v, sem).wait()
        pltpu.sync_copy(rows_v, agg_sh.at[dst_v.at[j]], add=True)
        return carry

    nk = jnp.where(cid == 0, K0, K1)
    lax.fori_loop(0, nk, chunk, 0)
    plsc.subcore_barrier()

    @pl.when(cid == 0)
    def _():
        pltpu.sync_copy(agg_sh.at[pl.ds(sid * RPT, RPT)],
                        out0.at[pl.ds(sid * RPT, RPT)])

    @pl.when(cid == 1)
    def _():
        pltpu.sync_copy(agg_sh.at[pl.ds(sid * RPT, RPT)],
                        out1.at[pl.ds(sid * RPT, RPT)])


_sc_agg = functools.partial(
    pl.kernel,
    mesh=plsc.VectorSubcoreMesh(core_axis_name="c", subcore_axis_name="s"),
    compiler_params=pltpu.CompilerParams(use_tc_tiling_on_sc=False),
    out_type=[jax.ShapeDtypeStruct((NP, D_H), jnp.float32),
              jax.ShapeDtypeStruct((NP, D_H), jnp.float32)],
    scratch_types=(
        [pltpu.VMEM((K0, CB), jnp.int32),
         pltpu.VMEM((K0, CB), jnp.int32),
         pltpu.VMEM((CB, D_H), jnp.float32),
         pltpu.VMEM_SHARED((NP, D_H), jnp.float32),
         pltpu.SemaphoreType.DMA]
    ),
)(_sc_agg_body)


BN = 2000


def _tc_body(p0_ref, p1_ref, x_ref, w_ref, om_ref, b_ref, hw_ref, hb_ref,
             out_ref):
    agg = p0_ref[...] + p1_ref[...]
    dn = (((1,), (1,)), ((), ()))
    t = lax.dot_general(agg, w_ref[...], dn, preferred_element_type=jnp.float32)
    t += lax.dot_general(x_ref[...], om_ref[...], dn,
                         preferred_element_type=jnp.float32)
    h = jnp.maximum(t + b_ref[...], 0.0)
    o = lax.dot_general(h, hw_ref[...], dn, preferred_element_type=jnp.float32)
    o += hb_ref[...]
    m = jnp.max(o, axis=-1, keepdims=True)
    ex = jnp.exp(o - m)
    s = jnp.sum(ex, axis=-1, keepdims=True)
    out_ref[...] = o - m - jnp.log(s)


def _tc_head(p0, p1, x, W, Omega, b, head_W, head_b):
    grid = (N // BN,)
    row_spec = pl.BlockSpec((BN, D_H), lambda i: (i, 0))
    return pl.pallas_call(
        _tc_body,
        grid=grid,
        in_specs=[
            row_spec,
            row_spec,
            row_spec,
            pl.BlockSpec((D_H, D_H), lambda i: (0, 0)),
            pl.BlockSpec((D_H, D_H), lambda i: (0, 0)),
            pl.BlockSpec((1, D_H), lambda i: (0, 0)),
            pl.BlockSpec((D_OUT, D_H), lambda i: (0, 0)),
            pl.BlockSpec((1, D_OUT), lambda i: (0, 0)),
        ],
        out_specs=pl.BlockSpec((BN, D_OUT), lambda i: (i, 0)),
        out_shape=jax.ShapeDtypeStruct((N, D_OUT), jnp.float32),
    )(p0, p1, x, W, Omega, b, head_W, head_b)


@jax.jit
def kernel(node_index, x, edge_index, emb_table, W, Omega, b, head_W, head_b):
    del node_index
    edges = edge_index.reshape(2, NR, CB)
    zeros = jnp.zeros((NP, D_H), jnp.float32)

    p0, p1 = _sc_agg(edges, emb_table, zeros)
    return _tc_head(p0, p1, x, W, Omega, b.reshape(1, D_H),
                    head_W, head_b.reshape(1, D_OUT))
